# baseline (speedup 1.0000x reference)
import numpy as np
import jax
import jax.numpy as jnp
from jax import lax
from jax.experimental import pallas as pl
from jax.experimental.pallas import tpu as pltpu

_BF = jnp.bfloat16
_NB = 16


def _edge_tile(e_pad):
    for te in (512, 256, 128):
        if e_pad % te == 0:
            return te
    return e_pad


def _conv1_extract_kernel(ea_ref, s_ref, m_ref, xb_ref,
                          w1a_ref, b1a_ref, w1b_ref, b1b_ref,
                          wr1_ref, bc1_ref, bd_ref,
                          h1_ref, idx_ref, acc_ref):
    t = pl.program_id(0)
    f32 = jnp.float32

    @pl.when(t == 0)
    def _init():
        acc_ref[...] = jnp.zeros_like(acc_ref)

    ea = ea_ref[...]
    w1a = w1a_ref[...]
    hid = jnp.maximum(ea[:, 0:1] * w1a[0:1, :] + ea[:, 1:2] * w1a[1:2, :]
                      + b1a_ref[...], 0.0)
    z = jnp.dot(hid, w1b_ref[...], preferred_element_type=f32) + b1b_ref[...]

    s = s_ref[...]
    m = m_ref[...]

    xgb = jnp.dot(s, xb_ref[...], preferred_element_type=f32)
    xg = xgb[:, 0:2]
    msg = xg[:, 0:1] * z[:, 0:32] + xg[:, 1:2] * z[:, 32:64]
    acc_ref[...] += jnp.dot(m, msg, preferred_element_type=f32)

    idx_ref[...] = xgb + lax.dot_general(m, bd_ref[...],
                                         (((0,), (0,)), ((), ())),
                                         preferred_element_type=f32)

    @pl.when(t == pl.num_programs(0) - 1)
    def _finalize():
        x = xb_ref[...][:, 0:2]
        wr = wr1_ref[...]
        root = x[:, 0:1] * wr[0:1, :] + x[:, 1:2] * wr[1:2, :]
        h1 = jnp.maximum(acc_ref[...] + root + bc1_ref[...], 0.0)
        h1_ref[...] = h1.astype(h1_ref.dtype)


def _conv2_head_kernel(ea_ref, idx_ref, hrs_ref,
                       w2a_ref, b2a_ref, w2b_ref, b2b_ref,
                       r2_ref, q2_ref, r16_ref, q16_ref, q16t_ref,
                       wr2b_ref, bc2t_ref, wf1b_ref, bf1t_ref,
                       wf2b_ref, bf2t_ref,
                       out_ref, acc_ref):
    t = pl.program_id(0)
    f32 = jnp.float32
    n_hi = acc_ref.shape[0]
    te = ea_ref.shape[0]

    @pl.when(t == 0)
    def _init():
        acc_ref[...] = jnp.zeros_like(acc_ref)

    ea = ea_ref[...]
    w2a = w2a_ref[...]
    hid = jnp.maximum(ea[:, 0:1] * w2a[0:1, :] + ea[:, 1:2] * w2a[1:2, :]
                      + b2a_ref[...], 0.0)
    z = (jnp.dot(hid.astype(_BF), w2b_ref[...], preferred_element_type=f32)
         + b2b_ref[...]).astype(_BF)

    idx = idx_ref[...]
    shi = jnp.round(idx[:, 2:3] * (1.0 / _NB))
    slo = jnp.round(idx[:, 3:4])
    w = idx[:, 6:7]
    winv = 1.0 / jnp.maximum(w, 1e-30)
    dhi = jnp.round(idx[:, 4:5] * winv * (1.0 / _NB))
    dlo = jnp.round(idx[:, 5:6] * winv)

    ihi = lax.broadcasted_iota(jnp.int32, (te, n_hi), 1).astype(f32)
    ilo = lax.broadcasted_iota(jnp.int32, (te, _NB), 1).astype(f32)
    oh_shi = (shi == ihi).astype(_BF)
    oh_slo = (slo == ilo).astype(_BF)
    oh_dhi = (dhi == ihi).astype(_BF)
    oh_dlow = ((dlo == ilo).astype(f32) * w).astype(_BF)

    hrs = hrs_ref[...]
    hb = jnp.dot(oh_shi, hrs, preferred_element_type=f32).astype(_BF)
    rep_slo = jnp.dot(oh_slo, r16_ref[...],
                      preferred_element_type=f32).astype(_BF)
    hg = jnp.dot(hb * rep_slo, q16_ref[...],
                 preferred_element_type=f32).astype(_BF)

    hg_rep = jnp.dot(hg, r2_ref[...],
                     preferred_element_type=f32).astype(_BF)
    msg = jnp.dot(hg_rep * z, q2_ref[...],
                  preferred_element_type=f32).astype(_BF)

    msg_t = jnp.dot(msg, q16t_ref[...],
                    preferred_element_type=f32).astype(_BF)
    rep_dlo = jnp.dot(oh_dlow, r16_ref[...],
                      preferred_element_type=f32).astype(_BF)
    acc_ref[...] += lax.dot_general(oh_dhi, rep_dlo * msg_t,
                                    (((0,), (0,)), ((), ())),
                                    preferred_element_type=f32)

    @pl.when(t == pl.num_programs(0) - 1)
    def _finalize():
        h2 = jnp.maximum(acc_ref[...]
                         + jnp.dot(hrs_ref[...], wr2b_ref[...],
                                   preferred_element_type=f32)
                         + bc2t_ref[...], 0.0)
        h3 = jnp.maximum(jnp.dot(h2.astype(_BF), wf1b_ref[...],
                                 preferred_element_type=f32)
                         + bf1t_ref[...], 0.0)
        out_ref[...] = (jnp.dot(h3.astype(_BF), wf2b_ref[...],
                                preferred_element_type=f32)
                        + bf2t_ref[...])


def _full(arr):
    nd = arr.ndim
    return pl.BlockSpec(arr.shape, lambda *_, _n=nd: (0,) * _n)


def kernel(x, edge_attr_pad, S, M,
           w1a, b1a, w1b, b1b, w2a, b2a, w2b, b2b,
           wr1, bc1, wr2, bc2, wfc1, bfc1, wfc2, bfc2, r2, q2):
    f32 = jnp.float32
    n = x.shape[0]
    e_pad = edge_attr_pad.shape[0]
    te = _edge_tile(e_pad)
    grid = (e_pad // te,)
    wide = 32 * _NB

    ar = np.arange(n)
    hi16 = (_NB * (ar // _NB)).astype(np.float32)
    lo = (ar % _NB).astype(np.float32)
    xcols = np.zeros((n, 6), np.float32)
    xcols[:, 0] = hi16
    xcols[:, 1] = lo
    bd = np.zeros((n, 8), np.float32)
    bd[:, 4] = hi16
    bd[:, 5] = lo
    bd[:, 6] = 1.0

    xb = jnp.concatenate([x, jnp.asarray(xcols)], axis=1)
    conv1_args = (edge_attr_pad, S, M, xb, w1a, b1a, w1b, b1b, wr1, bc1,
                  jnp.asarray(bd))
    h1, idx = pl.pallas_call(
        _conv1_extract_kernel,
        out_shape=[jax.ShapeDtypeStruct((n, 32), _BF),
                   jax.ShapeDtypeStruct((e_pad, 8), f32)],
        grid=grid,
        in_specs=[
            pl.BlockSpec((te, 2), lambda t: (t, 0)),
            pl.BlockSpec((te, n), lambda t: (t, 0)),
            pl.BlockSpec((n, te), lambda t: (0, t)),
        ] + [_full(a) for a in conv1_args[3:]],
        out_specs=[pl.BlockSpec((n, 32), lambda t: (0, 0)),
                   pl.BlockSpec((te, 8), lambda t: (t, 0))],
        scratch_shapes=[pltpu.VMEM((n, 32), f32)],
        compiler_params=pltpu.CompilerParams(
            dimension_semantics=("arbitrary",)),
    )(*conv1_args)

    n_hi = n // _NB
    h1_rs = h1.reshape(n_hi, wide)
    jw = np.arange(wide)
    r16 = jnp.asarray((jw[None, :] // 32 == np.arange(_NB)[:, None])
                      .astype(np.float32), dtype=_BF)
    q16 = jnp.asarray((jw[:, None] % 32 == np.arange(32)[None, :])
                      .astype(np.float32), dtype=_BF)
    q16t = jnp.asarray((jw[None, :] % 32 == np.arange(32)[:, None])
                       .astype(np.float32), dtype=_BF)
    eye16 = jnp.eye(_NB, dtype=f32)
    wr2b = jnp.kron(eye16, wr2).astype(_BF)
    wf1b = jnp.kron(eye16, wfc1).astype(_BF)
    wf2b = jnp.kron(eye16, wfc2).astype(_BF)
    bc2t = jnp.tile(bc2, (1, _NB))
    bf1t = jnp.tile(bfc1, (1, _NB))
    bf2t = jnp.tile(bfc2, (1, _NB))

    conv2_args = (edge_attr_pad, idx, h1_rs,
                  w2a, b2a, w2b.astype(_BF), b2b.astype(_BF),
                  r2.astype(_BF), q2.astype(_BF), r16, q16, q16t,
                  wr2b, bc2t, wf1b, bf1t, wf2b, bf2t)
    te2 = 4096 if e_pad % 4096 == 0 else te
    out2d = pl.pallas_call(
        _conv2_head_kernel,
        out_shape=jax.ShapeDtypeStruct((n_hi, 2 * _NB), f32),
        grid=(e_pad // te2,),
        in_specs=[
            pl.BlockSpec((te2, 2), lambda t: (t, 0)),
            pl.BlockSpec((te2, 8), lambda t: (t, 0)),
        ] + [_full(a) for a in conv2_args[2:]],
        out_specs=pl.BlockSpec((n_hi, 2 * _NB), lambda t: (0, 0)),
        scratch_shapes=[pltpu.VMEM((n_hi, wide), f32)],
        compiler_params=pltpu.CompilerParams(
            dimension_semantics=("arbitrary",)),
    )(*conv2_args)
    return out2d.reshape(n, 2)


# speedup vs baseline: 1.0201x; 1.0201x over previous
import numpy as np
import jax
import jax.numpy as jnp
from jax import lax
from jax.experimental import pallas as pl
from jax.experimental.pallas import tpu as pltpu

_BF = jnp.bfloat16
_NB = 16


def _edge_tile(e_pad):
    for te in (512, 256, 128):
        if e_pad % te == 0:
            return te
    return e_pad


def _conv1_extract_kernel(ea_ref, s_ref, m_ref, xb_ref,
                          w1a_ref, b1a_ref, w1b_ref, b1b_ref,
                          wr1_ref, bc1_ref, bd_ref,
                          h1_ref, idx_ref, acc_ref):
    t = pl.program_id(0)
    f32 = jnp.float32

    @pl.when(t == 0)
    def _init():
        acc_ref[...] = jnp.zeros_like(acc_ref)

    ea = ea_ref[...]
    w1a = w1a_ref[...]
    hid = jnp.maximum(ea[:, 0:1] * w1a[0:1, :] + ea[:, 1:2] * w1a[1:2, :]
                      + b1a_ref[...], 0.0)
    z = jnp.dot(hid, w1b_ref[...], preferred_element_type=f32) + b1b_ref[...]

    s = s_ref[...]
    m = m_ref[...]

    xgb = jnp.dot(s, xb_ref[...], preferred_element_type=f32)
    xg = xgb[:, 0:2]
    msg = xg[:, 0:1] * z[:, 0:32] + xg[:, 1:2] * z[:, 32:64]
    acc_ref[...] += jnp.dot(m, msg, preferred_element_type=f32)

    idx_ref[...] = xgb + lax.dot_general(m, bd_ref[...],
                                         (((0,), (0,)), ((), ())),
                                         preferred_element_type=f32)

    @pl.when(t == pl.num_programs(0) - 1)
    def _finalize():
        x = xb_ref[...][:, 0:2]
        wr = wr1_ref[...]
        root = x[:, 0:1] * wr[0:1, :] + x[:, 1:2] * wr[1:2, :]
        h1 = jnp.maximum(acc_ref[...] + root + bc1_ref[...], 0.0)
        h1_ref[...] = h1.astype(h1_ref.dtype)


def _conv2_head_kernel(ea_ref, idx_ref, hrs_ref,
                       w2a_ref, b2a_ref, w2b_ref, b2b_ref,
                       r2_ref, q2_ref, r16_ref, q16_ref, q16t_ref,
                       wr2b_ref, bc2t_ref, wf1b_ref, bf1t_ref,
                       wf2b_ref, bf2t_ref,
                       out_ref, acc_ref):
    t = pl.program_id(0)
    f32 = jnp.float32
    n_hi = acc_ref.shape[0]
    te = ea_ref.shape[0]

    @pl.when(t == 0)
    def _init():
        acc_ref[...] = jnp.zeros_like(acc_ref)

    ea = ea_ref[...]
    w2a = w2a_ref[...]
    hid = jnp.maximum(ea[:, 0:1] * w2a[0:1, :] + ea[:, 1:2] * w2a[1:2, :]
                      + b2a_ref[...], 0.0)
    z = (jnp.dot(hid.astype(_BF), w2b_ref[...], preferred_element_type=f32)
         + b2b_ref[...]).astype(_BF)

    idx = idx_ref[...]
    shi = jnp.round(idx[:, 2:3] * (1.0 / _NB))
    slo = jnp.round(idx[:, 3:4])
    w = idx[:, 6:7]
    winv = 1.0 / jnp.maximum(w, 1e-30)
    dhi = jnp.round(idx[:, 4:5] * winv * (1.0 / _NB))
    dlo = jnp.round(idx[:, 5:6] * winv)

    ihi = lax.broadcasted_iota(jnp.int32, (te, n_hi), 1).astype(f32)
    ilo = lax.broadcasted_iota(jnp.int32, (te, _NB), 1).astype(f32)
    oh_shi = (shi == ihi).astype(_BF)
    oh_slo = (slo == ilo).astype(_BF)
    oh_dhi = (dhi == ihi).astype(_BF)
    oh_dlow = ((dlo == ilo).astype(f32) * w).astype(_BF)

    hrs = hrs_ref[...]
    hb = jnp.dot(oh_shi, hrs, preferred_element_type=f32).astype(_BF)
    rep_slo = jnp.dot(oh_slo, r16_ref[...],
                      preferred_element_type=f32).astype(_BF)
    hg = jnp.dot(hb * rep_slo, q16_ref[...],
                 preferred_element_type=f32).astype(_BF)

    hg_rep = jnp.dot(hg, r2_ref[...],
                     preferred_element_type=f32).astype(_BF)
    msg = jnp.dot(hg_rep * z, q2_ref[...],
                  preferred_element_type=f32).astype(_BF)

    msg_t = jnp.dot(msg, q16t_ref[...],
                    preferred_element_type=f32).astype(_BF)
    rep_dlo = jnp.dot(oh_dlow, r16_ref[...],
                      preferred_element_type=f32).astype(_BF)
    acc_ref[...] += lax.dot_general(oh_dhi, rep_dlo * msg_t,
                                    (((0,), (0,)), ((), ())),
                                    preferred_element_type=f32)

    @pl.when(t == pl.num_programs(0) - 1)
    def _finalize():
        h2 = jnp.maximum(acc_ref[...]
                         + jnp.dot(hrs_ref[...], wr2b_ref[...],
                                   preferred_element_type=f32)
                         + bc2t_ref[...], 0.0)
        h3 = jnp.maximum(jnp.dot(h2.astype(_BF), wf1b_ref[...],
                                 preferred_element_type=f32)
                         + bf1t_ref[...], 0.0)
        out_ref[...] = (jnp.dot(h3.astype(_BF), wf2b_ref[...],
                                preferred_element_type=f32)
                        + bf2t_ref[...])


def _full(arr):
    nd = arr.ndim
    return pl.BlockSpec(arr.shape, lambda *_, _n=nd: (0,) * _n)


def kernel(x, edge_attr_pad, S, M,
           w1a, b1a, w1b, b1b, w2a, b2a, w2b, b2b,
           wr1, bc1, wr2, bc2, wfc1, bfc1, wfc2, bfc2, r2, q2):
    f32 = jnp.float32
    n = x.shape[0]
    e_pad = edge_attr_pad.shape[0]
    te = _edge_tile(e_pad)
    grid = (e_pad // te,)
    wide = 32 * _NB

    ar = np.arange(n)
    hi16 = (_NB * (ar // _NB)).astype(np.float32)
    lo = (ar % _NB).astype(np.float32)
    xcols = np.zeros((n, 6), np.float32)
    xcols[:, 0] = hi16
    xcols[:, 1] = lo
    bd = np.zeros((n, 8), np.float32)
    bd[:, 4] = hi16
    bd[:, 5] = lo
    bd[:, 6] = 1.0

    xb = jnp.concatenate([x, jnp.asarray(xcols)], axis=1)
    conv1_args = (edge_attr_pad, S, M, xb, w1a, b1a, w1b, b1b, wr1, bc1,
                  jnp.asarray(bd))
    h1, idx = pl.pallas_call(
        _conv1_extract_kernel,
        out_shape=[jax.ShapeDtypeStruct((n, 32), f32),
                   jax.ShapeDtypeStruct((e_pad, 8), f32)],
        grid=grid,
        in_specs=[
            pl.BlockSpec((te, 2), lambda t: (t, 0)),
            pl.BlockSpec((te, n), lambda t: (t, 0)),
            pl.BlockSpec((n, te), lambda t: (0, t)),
        ] + [_full(a) for a in conv1_args[3:]],
        out_specs=[pl.BlockSpec((n, 32), lambda t: (0, 0)),
                   pl.BlockSpec((te, 8), lambda t: (t, 0))],
        scratch_shapes=[pltpu.VMEM((n, 32), f32)],
        compiler_params=pltpu.CompilerParams(
            dimension_semantics=("arbitrary",)),
    )(*conv1_args)

    n_hi = n // _NB
    h1_rs = h1.reshape(n_hi, wide).astype(_BF)
    jw = np.arange(wide)
    r16 = jnp.asarray((jw[None, :] // 32 == np.arange(_NB)[:, None])
                      .astype(np.float32), dtype=_BF)
    q16 = jnp.asarray((jw[:, None] % 32 == np.arange(32)[None, :])
                      .astype(np.float32), dtype=_BF)
    q16t = jnp.asarray((jw[None, :] % 32 == np.arange(32)[:, None])
                       .astype(np.float32), dtype=_BF)
    eye16 = jnp.eye(_NB, dtype=f32)
    wr2b = jnp.kron(eye16, wr2).astype(_BF)
    wf1b = jnp.kron(eye16, wfc1).astype(_BF)
    wf2b = jnp.kron(eye16, wfc2).astype(_BF)
    bc2t = jnp.tile(bc2, (1, _NB))
    bf1t = jnp.tile(bfc1, (1, _NB))
    bf2t = jnp.tile(bfc2, (1, _NB))

    conv2_args = (edge_attr_pad, idx, h1_rs,
                  w2a, b2a, w2b.astype(_BF), b2b.astype(_BF),
                  r2.astype(_BF), q2.astype(_BF), r16, q16, q16t,
                  wr2b, bc2t, wf1b, bf1t, wf2b, bf2t)
    te2 = 4096 if e_pad % 4096 == 0 else te
    out2d = pl.pallas_call(
        _conv2_head_kernel,
        out_shape=jax.ShapeDtypeStruct((n_hi, 2 * _NB), f32),
        grid=(e_pad // te2,),
        in_specs=[
            pl.BlockSpec((te2, 2), lambda t: (t, 0)),
            pl.BlockSpec((te2, 8), lambda t: (t, 0)),
        ] + [_full(a) for a in conv2_args[2:]],
        out_specs=pl.BlockSpec((n_hi, 2 * _NB), lambda t: (0, 0)),
        scratch_shapes=[pltpu.VMEM((n_hi, wide), f32)],
        compiler_params=pltpu.CompilerParams(
            dimension_semantics=("arbitrary",)),
    )(*conv2_args)
    return out2d.reshape(n, 2)
